# Initial kernel scaffold; baseline (speedup 1.0000x reference)
import jax
import jax.numpy as jnp
from jax import lax
from jax.experimental import pallas as pl
from jax.experimental.pallas import tpu as pltpu
from jax.experimental.pallas import tpu_sc as plsc

N = 50000
E = 800000
H = 32
IN_DIM = 128
NCLS = 9
STEPS = 3

NB = 512
GN = 98
NP = NB * GN
NW = 32
EPW = 25600
EP = NW * EPW
CG = 128
NCH = EPW // CG
RING = 4
EB = 2048
GE = EP // EB
NSTRIPE = NP // 16
SPAN = 2560
NSPAN = EPW // SPAN
KPS = SPAN // CG

f32 = jnp.float32
i32 = jnp.int32

_SC_MESH = plsc.VectorSubcoreMesh(core_axis_name="c", subcore_axis_name="s")


def _gather_body(a_h, b_h, src_h, dst_h, ga_h, gb_h,
                 srcv, dstv, bufa, bufb, s0, s1, s2, s3):
    sems = (s0, s1, s2, s3)
    cid = lax.axis_index("c")
    sid = lax.axis_index("s")
    wid = sid * 2 + cid
    base = wid * EPW
    pltpu.sync_copy(src_h.at[pl.ds(base, EPW)], srcv)
    pltpu.sync_copy(dst_h.at[pl.ds(base, EPW)], dstv)

    def start(j, b):
        pltpu.async_copy(a_h.at[srcv.at[pl.ds(j * CG, CG)]], bufa.at[b], sems[b])
        pltpu.async_copy(b_h.at[dstv.at[pl.ds(j * CG, CG)]], bufb.at[b], sems[b])

    def wait(j, b):
        pltpu.make_async_copy(a_h.at[srcv.at[pl.ds(j * CG, CG)]], bufa.at[b], sems[b]).wait()
        pltpu.make_async_copy(b_h.at[dstv.at[pl.ds(j * CG, CG)]], bufb.at[b], sems[b]).wait()

    for b in range(RING):
        start(b, b)

    def body(jj, carry):
        for b in range(RING):
            j = jj * RING + b
            wait(j, b)
            pltpu.sync_copy(bufa.at[b], ga_h.at[pl.ds(base + j * CG, CG)])
            pltpu.sync_copy(bufb.at[b], gb_h.at[pl.ds(base + j * CG, CG)])

            @pl.when(j + RING < NCH)
            def _():
                start(j + RING, b)
        return carry

    lax.fori_loop(0, NCH // RING, body, 0)


def _sc_gather(a_t, b_t, src_p, dst_p):
    k = pl.kernel(
        _gather_body,
        out_type=[jax.ShapeDtypeStruct((EP, H), f32),
                  jax.ShapeDtypeStruct((EP, H), f32)],
        mesh=_SC_MESH,
        scratch_types=[
            pltpu.VMEM((EPW,), i32),
            pltpu.VMEM((EPW,), i32),
            pltpu.VMEM((RING, CG, H), f32),
            pltpu.VMEM((RING, CG, H), f32),
            pltpu.SemaphoreType.DMA,
            pltpu.SemaphoreType.DMA,
            pltpu.SemaphoreType.DMA,
            pltpu.SemaphoreType.DMA,
        ],
    )
    return k(a_t, b_t, src_p, dst_p)


def _scatter_body(msg_h, src3_h, zeros_h, agg0_h, agg1_h, shared, idxv, msgv):
    cid = lax.axis_index("c")
    sid = lax.axis_index("s")
    wid = sid * 2 + cid
    base = wid * EPW
    stripe = pl.ds(sid * NSTRIPE, NSTRIPE)
    pltpu.sync_copy(zeros_h.at[stripe], shared.at[stripe])
    plsc.subcore_barrier()
    pltpu.sync_copy(src3_h.at[wid], idxv)

    def body(sp, carry):
        pltpu.sync_copy(msg_h.at[pl.ds(base + sp * SPAN, SPAN)], msgv)

        def inner(kk, c2):
            j = sp * KPS + kk
            pltpu.sync_copy(msgv.at[pl.ds(kk * CG, CG)], shared.at[idxv.at[j]], add=True)
            return c2

        lax.fori_loop(0, KPS, inner, 0)
        return carry

    lax.fori_loop(0, NSPAN, body, 0)
    plsc.subcore_barrier()

    @pl.when(cid == 0)
    def _():
        pltpu.sync_copy(shared.at[stripe], agg0_h.at[stripe])

    @pl.when(cid == 1)
    def _():
        pltpu.sync_copy(shared.at[stripe], agg1_h.at[stripe])


def _sc_scatter(msg, src3, zeros_np):
    k = pl.kernel(
        _scatter_body,
        out_type=[jax.ShapeDtypeStruct((NP, H), f32),
                  jax.ShapeDtypeStruct((NP, H), f32)],
        mesh=_SC_MESH,
        scratch_types=[
            pltpu.VMEM_SHARED((NP, H), f32),
            pltpu.VMEM((NCH, 1, CG), i32),
            pltpu.VMEM((SPAN, H), f32),
        ],
    )
    return k(msg, src3, zeros_np)


def _sigm(x):
    return 1.0 / (1.0 + jnp.exp(-x))


def _dot(a, b):
    return jax.lax.dot_general(a, b, (((1,), (0,)), ((), ())),
                               preferred_element_type=f32)


def _pre_body(x_ref, pw0, pb0, pw1, pb1, pw2, pb2, pw3, pb3, w1s, w1d,
              x0_ref, a_ref, b_ref):
    h = x_ref[...]
    h = jnp.maximum(_dot(h, pw0[...]) + pb0[...], 0.0)
    h = jnp.maximum(_dot(h, pw1[...]) + pb1[...], 0.0)
    h = jnp.maximum(_dot(h, pw2[...]) + pb2[...], 0.0)
    h = _dot(h, pw3[...]) + pb3[...]
    x0_ref[...] = h
    a_ref[...] = _dot(h, w1s[...])
    b_ref[...] = _dot(h, w1d[...])


def _edge_body(ga_ref, gb_ref, b1, w2, b2, w3, b3, w4, b4, out_ref):
    h = jnp.maximum(ga_ref[...] + gb_ref[...] + b1[...], 0.0)
    h = jnp.maximum(_dot(h, w2[...]) + b2[...], 0.0)
    h = jnp.maximum(_dot(h, w3[...]) + b3[...], 0.0)
    out_ref[...] = _dot(h, w4[...]) + b4[...]


def _node_body(a0_ref, a1_ref, x0_ref, c_ref, h_ref, tgt_ref,
               qw0a, qw0b, qb0, qw1, qb1, qw2, qb2, qw3, qb3,
               wlx, wlh, bl, w1s, w1d, wo, bo,
               cout, hout, aout, bout, loss_ref):
    i = pl.program_id(0)
    agg = a0_ref[...] + a1_ref[...]
    x0 = x0_ref[...]
    u = jnp.maximum(_dot(agg, qw0a[...]) + _dot(x0, qw0b[...]) + qb0[...], 0.0)
    u = jnp.maximum(_dot(u, qw1[...]) + qb1[...], 0.0)
    u = jnp.maximum(_dot(u, qw2[...]) + qb2[...], 0.0)
    xc = _dot(u, qw3[...]) + qb3[...]
    z = _dot(xc, wlx[...]) + _dot(h_ref[...], wlh[...]) + bl[...]
    zi = z[:, 0:H]
    zj = z[:, H:2 * H]
    zf = z[:, 2 * H:3 * H]
    zo = z[:, 3 * H:4 * H]
    cn = c_ref[...] * _sigm(zf + 1.0) + _sigm(zi) * jnp.tanh(zj)
    hn = _sigm(zo) * jnp.tanh(cn)
    cout[...] = cn
    hout[...] = hn
    aout[...] = _dot(hn, w1s[...])
    bout[...] = _dot(hn, w1d[...])
    logits = _dot(hn, wo[...]) + bo[...]
    m = jnp.max(logits, axis=1, keepdims=True)
    lse = m + jnp.log(jnp.sum(jnp.exp(logits - m), axis=1, keepdims=True))
    t2 = tgt_ref[...].reshape(NB, 1)
    onehot = (t2 == lax.broadcasted_iota(i32, (NB, NCLS), 1)).astype(f32)
    picked = jnp.sum(logits * onehot, axis=1, keepdims=True)
    ce = lse - picked
    rows = i * NB + lax.broadcasted_iota(i32, (NB, 1), 0)
    part = jnp.sum(jnp.where(rows < N, ce, 0.0))

    @pl.when(i == 0)
    def _():
        loss_ref[0, 0] = part

    @pl.when(i > 0)
    def _():
        loss_ref[0, 0] = loss_ref[0, 0] + part


def _full(shape):
    return pl.BlockSpec(shape, lambda *_: tuple(0 for _ in shape))


def _tc_pre(xp, pre_ws, w1s, w1d):
    specs = [pl.BlockSpec((NB, IN_DIM), lambda i: (i, 0))]
    args = [xp]
    for (w, b) in pre_ws:
        specs += [_full(w.shape), _full((1, H))]
        args += [w, b.reshape(1, H)]
    specs += [_full((H, H)), _full((H, H))]
    args += [w1s, w1d]
    out = pl.pallas_call(
        _pre_body,
        grid=(GN,),
        in_specs=specs,
        out_specs=[pl.BlockSpec((NB, H), lambda i: (i, 0))] * 3,
        out_shape=[jax.ShapeDtypeStruct((NP, H), f32)] * 3,
    )(*args)
    return out


def _tc_edge(ga, gb, msg_ws):
    (w2, b2), (w3, b3), (w4, b4) = msg_ws[1], msg_ws[2], msg_ws[3]
    b1 = msg_ws[0][1]
    specs = [pl.BlockSpec((EB, H), lambda i: (i, 0))] * 2
    specs += [_full((1, H)), _full((H, H)), _full((1, H)),
              _full((H, H)), _full((1, H)), _full((H, H)), _full((1, H))]
    out = pl.pallas_call(
        _edge_body,
        grid=(GE,),
        in_specs=specs,
        out_specs=pl.BlockSpec((EB, H), lambda i: (i, 0)),
        out_shape=jax.ShapeDtypeStruct((EP, H), f32),
    )(ga, gb, b1.reshape(1, H), w2, b2.reshape(1, H),
      w3, b3.reshape(1, H), w4, b4.reshape(1, H))
    return out


def _tc_node(a0, a1, x0, c_st, h_st, tp, post_ws, lstm_w, lstm_b, w1s, w1d, wo, bo):
    (q0, qb0), (q1, qb1), (q2, qb2), (q3, qb3) = post_ws
    nodeblk = pl.BlockSpec((NB, H), lambda i: (i, 0))
    specs = [nodeblk] * 5 + [pl.BlockSpec((1, 1, NB), lambda i: (i, 0, 0))]
    specs += [_full((H, H)), _full((H, H)), _full((1, H)),
              _full((H, H)), _full((1, H)), _full((H, H)), _full((1, H)),
              _full((H, H)), _full((1, H)),
              _full((H, 4 * H)), _full((H, 4 * H)), _full((1, 4 * H)),
              _full((H, H)), _full((H, H)),
              _full((H, NCLS)), _full((1, NCLS))]
    out = pl.pallas_call(
        _node_body,
        grid=(GN,),
        in_specs=specs,
        out_specs=[nodeblk] * 4 + [pl.BlockSpec((1, 1), lambda i: (0, 0))],
        out_shape=[jax.ShapeDtypeStruct((NP, H), f32)] * 4
        + [jax.ShapeDtypeStruct((1, 1), f32)],
        compiler_params=pltpu.CompilerParams(
            dimension_semantics=("arbitrary",)),
    )(a0, a1, x0, c_st, h_st, tp,
      q0[:H], q0[H:], qb0.reshape(1, H),
      q1, qb1.reshape(1, H), q2, qb2.reshape(1, H), q3, qb3.reshape(1, H),
      lstm_w[:H], lstm_w[H:], lstm_b.reshape(1, 4 * H),
      w1s, w1d, wo, bo.reshape(1, NCLS))
    return out


def kernel(x, edge_index, edge_attr, targets, params):
    del edge_attr
    p = params
    src = edge_index[0].astype(i32)
    dst = edge_index[1].astype(i32)
    pad = jnp.full((EP - E,), N, i32)
    src_p = jnp.concatenate([src, pad])
    dst_p = jnp.concatenate([dst, pad])
    src3 = src_p.reshape(NW, NCH, 1, CG)
    xp = jnp.pad(x, ((0, NP - N), (0, 0)))
    tp = jnp.pad(targets.astype(i32), (0, NP - N)).reshape(GN, 1, NB)
    zeros_np = jnp.zeros((NP, H), f32)

    w1 = p["msg"][0][0]
    w1s = w1[:H]
    w1d = w1[H:2 * H]

    x0, a_t, b_t = _tc_pre(xp, p["pre"], w1s, w1d)
    c_st = jnp.zeros((NP, H), f32)
    h_st = jnp.zeros((NP, H), f32)
    losses = []
    for _ in range(STEPS):
        ga, gb = _sc_gather(a_t, b_t, src_p, dst_p)
        msg = _tc_edge(ga, gb, p["msg"])
        agg0, agg1 = _sc_scatter(msg, src3, zeros_np)
        c_st, h_st, a_t, b_t, lsum = _tc_node(
            agg0, agg1, x0, c_st, h_st, tp, p["post"],
            p["lstm_W"], p["lstm_b"], w1s, w1d, p["out_W"], p["out_b"])
        losses.append(lsum[0, 0] / (N * jnp.log(2.0)))
    return jnp.mean(jnp.stack(losses))


# speedup vs baseline: 2.7107x; 2.7107x over previous
import jax
import jax.numpy as jnp
from jax import lax
from jax.experimental import pallas as pl
from jax.experimental.pallas import tpu as pltpu
from jax.experimental.pallas import tpu_sc as plsc

N = 50000
E = 800000
H = 32
IN_DIM = 128
NCLS = 9
STEPS = 3

NB = 512
GN = 98
NP = NB * GN
NW = 32
EPW = 25600
EP = NW * EPW
CG = 128
NCH = EPW // CG
RING = 4
EB = 2048
GE = EP // EB
NSTRIPE = NP // 16
SPAN = 512
NSPAN = EPW // SPAN
KPS = SPAN // CG

f32 = jnp.float32
i32 = jnp.int32

_SC_MESH = plsc.VectorSubcoreMesh(core_axis_name="c", subcore_axis_name="s")


def _gather_body(a_h, b_h, src_h, dst_h, ga_h, gb_h,
                 srcv, dstv, bufa, bufb, s0, s1, s2, s3):
    sems = (s0, s1, s2, s3)
    cid = lax.axis_index("c")
    sid = lax.axis_index("s")
    wid = sid * 2 + cid
    base = wid * EPW
    pltpu.sync_copy(src_h.at[pl.ds(base, EPW)], srcv)
    pltpu.sync_copy(dst_h.at[pl.ds(base, EPW)], dstv)

    def start(j, b):
        pltpu.async_copy(a_h.at[srcv.at[pl.ds(j * CG, CG)]], bufa.at[b], sems[b])
        pltpu.async_copy(b_h.at[dstv.at[pl.ds(j * CG, CG)]], bufb.at[b], sems[b])

    def wait(j, b):
        pltpu.make_async_copy(a_h.at[srcv.at[pl.ds(j * CG, CG)]], bufa.at[b], sems[b]).wait()
        pltpu.make_async_copy(b_h.at[dstv.at[pl.ds(j * CG, CG)]], bufb.at[b], sems[b]).wait()

    for b in range(RING):
        start(b, b)

    def body(jj, carry):
        for b in range(RING):
            j = jj * RING + b
            wait(j, b)
            pltpu.sync_copy(bufa.at[b], ga_h.at[pl.ds(base + j * CG, CG)])
            pltpu.sync_copy(bufb.at[b], gb_h.at[pl.ds(base + j * CG, CG)])

            @pl.when(j + RING < NCH)
            def _():
                start(j + RING, b)
        return carry

    lax.fori_loop(0, NCH // RING, body, 0)


def _sc_gather(a_t, b_t, src_p, dst_p):
    k = pl.kernel(
        _gather_body,
        out_type=[jax.ShapeDtypeStruct((EP, H), f32),
                  jax.ShapeDtypeStruct((EP, H), f32)],
        mesh=_SC_MESH,
        scratch_types=[
            pltpu.VMEM((EPW,), i32),
            pltpu.VMEM((EPW,), i32),
            pltpu.VMEM((RING, CG, H), f32),
            pltpu.VMEM((RING, CG, H), f32),
            pltpu.SemaphoreType.DMA,
            pltpu.SemaphoreType.DMA,
            pltpu.SemaphoreType.DMA,
            pltpu.SemaphoreType.DMA,
        ],
        compiler_params=pltpu.CompilerParams(use_tc_tiling_on_sc=False),
    )
    return k(a_t, b_t, src_p, dst_p)


def _scatter_body(msg_h, src3_h, zeros_h, agg0_h, agg1_h, shared, idxb, msgv):
    cid = lax.axis_index("c")
    sid = lax.axis_index("s")
    wid = sid * 2 + cid
    base = wid * EPW
    stripe = pl.ds(sid * NSTRIPE, NSTRIPE)
    pltpu.sync_copy(zeros_h.at[stripe], shared.at[stripe])
    plsc.subcore_barrier()

    def body(sp, carry):
        pltpu.sync_copy(msg_h.at[pl.ds(base + sp * SPAN, SPAN)], msgv)

        def inner(kk, c2):
            j = sp * KPS + kk
            pltpu.sync_copy(src3_h.at[wid, j], idxb)
            pltpu.sync_copy(msgv.at[pl.ds(kk * CG, CG)], shared.at[idxb], add=True)
            return c2

        lax.fori_loop(0, KPS, inner, 0)
        return carry

    lax.fori_loop(0, NSPAN, body, 0)
    plsc.subcore_barrier()

    @pl.when(cid == 0)
    def _():
        pltpu.sync_copy(shared.at[stripe], agg0_h.at[stripe])

    @pl.when(cid == 1)
    def _():
        pltpu.sync_copy(shared.at[stripe], agg1_h.at[stripe])


def _sc_scatter(msg, src3, zeros_np):
    k = pl.kernel(
        _scatter_body,
        out_type=[jax.ShapeDtypeStruct((NP, H), f32),
                  jax.ShapeDtypeStruct((NP, H), f32)],
        mesh=_SC_MESH,
        scratch_types=[
            pltpu.VMEM_SHARED((NP, H), f32),
            pltpu.VMEM((CG,), i32),
            pltpu.VMEM((SPAN, H), f32),
        ],
        compiler_params=pltpu.CompilerParams(use_tc_tiling_on_sc=False),
    )
    return k(msg, src3, zeros_np)


def _sigm(x):
    return 1.0 / (1.0 + jnp.exp(-x))


def _dot(a, b):
    return jax.lax.dot_general(a, b, (((1,), (0,)), ((), ())),
                               preferred_element_type=f32)


def _pre_body(x_ref, pw0, pb0, pw1, pb1, pw2, pb2, pw3, pb3, w1s, w1d,
              x0_ref, a_ref, b_ref):
    h = x_ref[...]
    h = jnp.maximum(_dot(h, pw0[...]) + pb0[...], 0.0)
    h = jnp.maximum(_dot(h, pw1[...]) + pb1[...], 0.0)
    h = jnp.maximum(_dot(h, pw2[...]) + pb2[...], 0.0)
    h = _dot(h, pw3[...]) + pb3[...]
    x0_ref[...] = h
    a_ref[...] = _dot(h, w1s[...])
    b_ref[...] = _dot(h, w1d[...])


def _edge_body(ga_ref, gb_ref, b1, w2, b2, w3, b3, w4, b4, out_ref):
    h = jnp.maximum(ga_ref[...] + gb_ref[...] + b1[...], 0.0)
    h = jnp.maximum(_dot(h, w2[...]) + b2[...], 0.0)
    h = jnp.maximum(_dot(h, w3[...]) + b3[...], 0.0)
    out_ref[...] = _dot(h, w4[...]) + b4[...]


def _node_body(a0_ref, a1_ref, x0_ref, c_ref, h_ref, tgt_ref,
               qw0a, qw0b, qb0, qw1, qb1, qw2, qb2, qw3, qb3,
               wlx, wlh, bl, w1s, w1d, wo, bo,
               cout, hout, aout, bout, loss_ref):
    i = pl.program_id(0)
    agg = a0_ref[...] + a1_ref[...]
    x0 = x0_ref[...]
    u = jnp.maximum(_dot(agg, qw0a[...]) + _dot(x0, qw0b[...]) + qb0[...], 0.0)
    u = jnp.maximum(_dot(u, qw1[...]) + qb1[...], 0.0)
    u = jnp.maximum(_dot(u, qw2[...]) + qb2[...], 0.0)
    xc = _dot(u, qw3[...]) + qb3[...]
    z = _dot(xc, wlx[...]) + _dot(h_ref[...], wlh[...]) + bl[...]
    zi = z[:, 0:H]
    zj = z[:, H:2 * H]
    zf = z[:, 2 * H:3 * H]
    zo = z[:, 3 * H:4 * H]
    cn = c_ref[...] * _sigm(zf + 1.0) + _sigm(zi) * jnp.tanh(zj)
    hn = _sigm(zo) * jnp.tanh(cn)
    cout[...] = cn
    hout[...] = hn
    aout[...] = _dot(hn, w1s[...])
    bout[...] = _dot(hn, w1d[...])
    logits = _dot(hn, wo[...]) + bo[...]
    m = jnp.max(logits, axis=1, keepdims=True)
    lse = m + jnp.log(jnp.sum(jnp.exp(logits - m), axis=1, keepdims=True))
    t2 = tgt_ref[...].reshape(NB, 1)
    onehot = (t2 == lax.broadcasted_iota(i32, (NB, NCLS), 1)).astype(f32)
    picked = jnp.sum(logits * onehot, axis=1, keepdims=True)
    ce = lse - picked
    rows = i * NB + lax.broadcasted_iota(i32, (NB, 1), 0)
    part = jnp.sum(jnp.where(rows < N, ce, 0.0))

    @pl.when(i == 0)
    def _():
        loss_ref[...] = part.reshape(1, 1)

    @pl.when(i > 0)
    def _():
        loss_ref[...] = loss_ref[...] + part.reshape(1, 1)


def _full(shape):
    return pl.BlockSpec(shape, lambda *_: tuple(0 for _ in shape))


def _tc_pre(xp, pre_ws, w1s, w1d):
    specs = [pl.BlockSpec((NB, IN_DIM), lambda i: (i, 0))]
    args = [xp]
    for (w, b) in pre_ws:
        specs += [_full(w.shape), _full((1, H))]
        args += [w, b.reshape(1, H)]
    specs += [_full((H, H)), _full((H, H))]
    args += [w1s, w1d]
    out = pl.pallas_call(
        _pre_body,
        grid=(GN,),
        in_specs=specs,
        out_specs=[pl.BlockSpec((NB, H), lambda i: (i, 0))] * 3,
        out_shape=[jax.ShapeDtypeStruct((NP, H), f32)] * 3,
    )(*args)
    return out


def _tc_edge(ga, gb, msg_ws):
    (w2, b2), (w3, b3), (w4, b4) = msg_ws[1], msg_ws[2], msg_ws[3]
    b1 = msg_ws[0][1]
    specs = [pl.BlockSpec((EB, H), lambda i: (i, 0))] * 2
    specs += [_full((1, H)), _full((H, H)), _full((1, H)),
              _full((H, H)), _full((1, H)), _full((H, H)), _full((1, H))]
    out = pl.pallas_call(
        _edge_body,
        grid=(GE,),
        in_specs=specs,
        out_specs=pl.BlockSpec((EB, H), lambda i: (i, 0)),
        out_shape=jax.ShapeDtypeStruct((EP, H), f32),
    )(ga, gb, b1.reshape(1, H), w2, b2.reshape(1, H),
      w3, b3.reshape(1, H), w4, b4.reshape(1, H))
    return out


def _tc_node(a0, a1, x0, c_st, h_st, tp, post_ws, lstm_w, lstm_b, w1s, w1d, wo, bo):
    (q0, qb0), (q1, qb1), (q2, qb2), (q3, qb3) = post_ws
    nodeblk = pl.BlockSpec((NB, H), lambda i: (i, 0))
    specs = [nodeblk] * 5 + [pl.BlockSpec((1, 1, NB), lambda i: (i, 0, 0))]
    specs += [_full((H, H)), _full((H, H)), _full((1, H)),
              _full((H, H)), _full((1, H)), _full((H, H)), _full((1, H)),
              _full((H, H)), _full((1, H)),
              _full((H, 4 * H)), _full((H, 4 * H)), _full((1, 4 * H)),
              _full((H, H)), _full((H, H)),
              _full((H, NCLS)), _full((1, NCLS))]
    out = pl.pallas_call(
        _node_body,
        grid=(GN,),
        in_specs=specs,
        out_specs=[nodeblk] * 4 + [pl.BlockSpec((1, 1), lambda i: (0, 0))],
        out_shape=[jax.ShapeDtypeStruct((NP, H), f32)] * 4
        + [jax.ShapeDtypeStruct((1, 1), f32)],
        compiler_params=pltpu.CompilerParams(
            dimension_semantics=("arbitrary",)),
    )(a0, a1, x0, c_st, h_st, tp,
      q0[:H], q0[H:], qb0.reshape(1, H),
      q1, qb1.reshape(1, H), q2, qb2.reshape(1, H), q3, qb3.reshape(1, H),
      lstm_w[:H], lstm_w[H:], lstm_b.reshape(1, 4 * H),
      w1s, w1d, wo, bo.reshape(1, NCLS))
    return out


def kernel(x, edge_index, edge_attr, targets, params):
    del edge_attr
    p = params
    src = edge_index[0].astype(i32)
    dst = edge_index[1].astype(i32)
    pad = jnp.full((EP - E,), N, i32)
    src_p = jnp.concatenate([src, pad])
    dst_p = jnp.concatenate([dst, pad])
    src3 = src_p.reshape(NW, NCH, CG)
    xp = jnp.pad(x, ((0, NP - N), (0, 0)))
    tp = jnp.pad(targets.astype(i32), (0, NP - N)).reshape(GN, 1, NB)
    zeros_np = jnp.zeros((NP, H), f32)

    w1 = p["msg"][0][0]
    w1s = w1[:H]
    w1d = w1[H:2 * H]

    x0, a_t, b_t = _tc_pre(xp, p["pre"], w1s, w1d)
    c_st = jnp.zeros((NP, H), f32)
    h_st = jnp.zeros((NP, H), f32)
    losses = []
    for _ in range(STEPS):
        ga, gb = _sc_gather(a_t, b_t, src_p, dst_p)
        msg = _tc_edge(ga, gb, p["msg"])
        agg0, agg1 = _sc_scatter(msg, src3, zeros_np)
        c_st, h_st, a_t, b_t, lsum = _tc_node(
            agg0, agg1, x0, c_st, h_st, tp, p["post"],
            p["lstm_W"], p["lstm_b"], w1s, w1d, p["out_W"], p["out_b"])
        losses.append(lsum[0, 0] / (N * jnp.log(2.0)))
    return jnp.mean(jnp.stack(losses))


# speedup vs baseline: 5.6747x; 2.0934x over previous
import jax
import jax.numpy as jnp
from jax import lax
from jax.experimental import pallas as pl
from jax.experimental.pallas import tpu as pltpu
from jax.experimental.pallas import tpu_sc as plsc

N = 50000
E = 800000
H = 32
IN_DIM = 128
NCLS = 9
STEPS = 3

NB = 512
GN = 98
NP = NB * GN
NW = 32
EPW = 25600
EP = NW * EPW
CG = 128
NCH = EPW // CG
RING = 4
EPP = EP * H // (4 * H)
EBP = 2048
GEP = EPP // EBP
NSTRIPE = NP // 16
SPAN = 512
NSPAN = EPW // SPAN
KPS = SPAN // CG

f32 = jnp.float32
i32 = jnp.int32

_SC_MESH = plsc.VectorSubcoreMesh(core_axis_name="c", subcore_axis_name="s")


def _gather_body(a_h, b_h, src_h, dst_h, ga_h, gb_h,
                 srcv, dstv, bufa, bufb, s0, s1, s2, s3):
    sems = (s0, s1, s2, s3)
    cid = lax.axis_index("c")
    sid = lax.axis_index("s")
    wid = sid * 2 + cid
    base = wid * EPW
    pltpu.sync_copy(src_h.at[pl.ds(base, EPW)], srcv)
    pltpu.sync_copy(dst_h.at[pl.ds(base, EPW)], dstv)

    def start(j, b):
        pltpu.async_copy(a_h.at[srcv.at[pl.ds(j * CG, CG)]], bufa.at[b], sems[b])
        pltpu.async_copy(b_h.at[dstv.at[pl.ds(j * CG, CG)]], bufb.at[b], sems[b])

    def wait(j, b):
        pltpu.make_async_copy(a_h.at[srcv.at[pl.ds(j * CG, CG)]], bufa.at[b], sems[b]).wait()
        pltpu.make_async_copy(b_h.at[dstv.at[pl.ds(j * CG, CG)]], bufb.at[b], sems[b]).wait()

    for b in range(RING):
        start(b, b)

    def body(jj, carry):
        for b in range(RING):
            j = jj * RING + b
            wait(j, b)
            pltpu.sync_copy(bufa.at[b], ga_h.at[pl.ds(base + j * CG, CG)])
            pltpu.sync_copy(bufb.at[b], gb_h.at[pl.ds(base + j * CG, CG)])

            @pl.when(j + RING < NCH)
            def _():
                start(j + RING, b)
        return carry

    lax.fori_loop(0, NCH // RING, body, 0)


def _sc_gather(a_t, b_t, src_p, dst_p):
    k = pl.kernel(
        _gather_body,
        out_type=[jax.ShapeDtypeStruct((EP, H), f32),
                  jax.ShapeDtypeStruct((EP, H), f32)],
        mesh=_SC_MESH,
        scratch_types=[
            pltpu.VMEM((EPW,), i32),
            pltpu.VMEM((EPW,), i32),
            pltpu.VMEM((RING, CG, H), f32),
            pltpu.VMEM((RING, CG, H), f32),
            pltpu.SemaphoreType.DMA,
            pltpu.SemaphoreType.DMA,
            pltpu.SemaphoreType.DMA,
            pltpu.SemaphoreType.DMA,
        ],
        compiler_params=pltpu.CompilerParams(use_tc_tiling_on_sc=False),
    )
    return k(a_t, b_t, src_p, dst_p)


def _scatter_body(msg_h, src3_h, zeros_h, agg0_h, agg1_h, shared, idxb, msgv):
    cid = lax.axis_index("c")
    sid = lax.axis_index("s")
    wid = sid * 2 + cid
    base = wid * EPW
    stripe = pl.ds(sid * NSTRIPE, NSTRIPE)
    pltpu.sync_copy(zeros_h.at[stripe], shared.at[stripe])
    plsc.subcore_barrier()

    def body(sp, carry):
        pltpu.sync_copy(msg_h.at[pl.ds(base + sp * SPAN, SPAN)], msgv)

        def inner(kk, c2):
            j = sp * KPS + kk
            pltpu.sync_copy(src3_h.at[wid, j], idxb)
            pltpu.sync_copy(msgv.at[pl.ds(kk * CG, CG)], shared.at[idxb], add=True)
            return c2

        lax.fori_loop(0, KPS, inner, 0)
        return carry

    lax.fori_loop(0, NSPAN, body, 0)
    plsc.subcore_barrier()

    @pl.when(cid == 0)
    def _():
        pltpu.sync_copy(shared.at[stripe], agg0_h.at[stripe])

    @pl.when(cid == 1)
    def _():
        pltpu.sync_copy(shared.at[stripe], agg1_h.at[stripe])


def _sc_scatter(msg, src3, zeros_np):
    k = pl.kernel(
        _scatter_body,
        out_type=[jax.ShapeDtypeStruct((NP, H), f32),
                  jax.ShapeDtypeStruct((NP, H), f32)],
        mesh=_SC_MESH,
        scratch_types=[
            pltpu.VMEM_SHARED((NP, H), f32),
            pltpu.VMEM((CG,), i32),
            pltpu.VMEM((SPAN, H), f32),
        ],
        compiler_params=pltpu.CompilerParams(use_tc_tiling_on_sc=False),
    )
    return k(msg, src3, zeros_np)


def _sigm(x):
    return 1.0 / (1.0 + jnp.exp(-x))


def _dot(a, b):
    return jax.lax.dot_general(a, b, (((1,), (0,)), ((), ())),
                               preferred_element_type=f32)


def _pre_body(x_ref, pw0, pb0, pw1, pb1, pw2, pb2, pw3, pb3, w1s, w1d,
              x0_ref, a_ref, b_ref):
    h = x_ref[...]
    h = jnp.maximum(_dot(h, pw0[...]) + pb0[...], 0.0)
    h = jnp.maximum(_dot(h, pw1[...]) + pb1[...], 0.0)
    h = jnp.maximum(_dot(h, pw2[...]) + pb2[...], 0.0)
    h = _dot(h, pw3[...]) + pb3[...]
    x0_ref[...] = h
    a_ref[...] = _dot(h, w1s[...])
    b_ref[...] = _dot(h, w1d[...])


def _edge_body(ga_ref, gb_ref, b1, w2, b2, w3, b3, w4, b4, out_ref):
    h = jnp.maximum(ga_ref[...] + gb_ref[...] + b1[...], 0.0)
    h = jnp.maximum(_dot(h.astype(jnp.bfloat16), w2[...]) + b2[...], 0.0)
    h = jnp.maximum(_dot(h.astype(jnp.bfloat16), w3[...]) + b3[...], 0.0)
    out_ref[...] = _dot(h.astype(jnp.bfloat16), w4[...]) + b4[...]


def _node_body(a0_ref, a1_ref, x0_ref, c_ref, h_ref, tgt_ref,
               qw0a, qw0b, qb0, qw1, qb1, qw2, qb2, qw3, qb3,
               wlx, wlh, bl, w1s, w1d, wo, bo,
               cout, hout, aout, bout, loss_ref):
    i = pl.program_id(0)
    agg = a0_ref[...] + a1_ref[...]
    x0 = x0_ref[...]
    u = jnp.maximum(_dot(agg, qw0a[...]) + _dot(x0, qw0b[...]) + qb0[...], 0.0)
    u = jnp.maximum(_dot(u, qw1[...]) + qb1[...], 0.0)
    u = jnp.maximum(_dot(u, qw2[...]) + qb2[...], 0.0)
    xc = _dot(u, qw3[...]) + qb3[...]
    z = _dot(xc, wlx[...]) + _dot(h_ref[...], wlh[...]) + bl[...]
    zi = z[:, 0:H]
    zj = z[:, H:2 * H]
    zf = z[:, 2 * H:3 * H]
    zo = z[:, 3 * H:4 * H]
    cn = c_ref[...] * _sigm(zf + 1.0) + _sigm(zi) * jnp.tanh(zj)
    hn = _sigm(zo) * jnp.tanh(cn)
    cout[...] = cn
    hout[...] = hn
    aout[...] = _dot(hn, w1s[...])
    bout[...] = _dot(hn, w1d[...])
    logits = _dot(hn, wo[...]) + bo[...]
    m = jnp.max(logits, axis=1, keepdims=True)
    lse = m + jnp.log(jnp.sum(jnp.exp(logits - m), axis=1, keepdims=True))
    t2 = tgt_ref[...].reshape(NB, 1)
    onehot = (t2 == lax.broadcasted_iota(i32, (NB, NCLS), 1)).astype(f32)
    picked = jnp.sum(logits * onehot, axis=1, keepdims=True)
    ce = lse - picked
    rows = i * NB + lax.broadcasted_iota(i32, (NB, 1), 0)
    part = jnp.sum(jnp.where(rows < N, ce, 0.0))

    @pl.when(i == 0)
    def _():
        loss_ref[...] = part.reshape(1, 1)

    @pl.when(i > 0)
    def _():
        loss_ref[...] = loss_ref[...] + part.reshape(1, 1)


def _full(shape):
    return pl.BlockSpec(shape, lambda *_: tuple(0 for _ in shape))


def _tc_pre(xp, pre_ws, w1s, w1d):
    specs = [pl.BlockSpec((NB, IN_DIM), lambda i: (i, 0))]
    args = [xp]
    for (w, b) in pre_ws:
        specs += [_full(w.shape), _full((1, H))]
        args += [w, b.reshape(1, H)]
    specs += [_full((H, H)), _full((H, H))]
    args += [w1s, w1d]
    out = pl.pallas_call(
        _pre_body,
        grid=(GN,),
        in_specs=specs,
        out_specs=[pl.BlockSpec((NB, H), lambda i: (i, 0))] * 3,
        out_shape=[jax.ShapeDtypeStruct((NP, H), f32)] * 3,
    )(*args)
    return out


def _bd4(w):
    z = jnp.zeros((4 * H, 4 * H), w.dtype)
    for k in range(4):
        z = z.at[k * H:(k + 1) * H, k * H:(k + 1) * H].set(w)
    return z


def _tc_edge(ga_p, gb_p, msg_ws):
    (w2, b2), (w3, b3), (w4, b4) = msg_ws[1], msg_ws[2], msg_ws[3]
    b1 = msg_ws[0][1]
    bf16 = jnp.bfloat16
    specs = [pl.BlockSpec((EBP, 4 * H), lambda i: (i, 0))] * 2
    specs += [_full((1, 4 * H)), _full((4 * H, 4 * H)), _full((1, 4 * H)),
              _full((4 * H, 4 * H)), _full((1, 4 * H)),
              _full((4 * H, 4 * H)), _full((1, 4 * H))]
    out = pl.pallas_call(
        _edge_body,
        grid=(GEP,),
        in_specs=specs,
        out_specs=pl.BlockSpec((EBP, 4 * H), lambda i: (i, 0)),
        out_shape=jax.ShapeDtypeStruct((EPP, 4 * H), f32),
    )(ga_p, gb_p,
      jnp.tile(b1, 4).reshape(1, 4 * H),
      _bd4(w2).astype(bf16), jnp.tile(b2, 4).reshape(1, 4 * H),
      _bd4(w3).astype(bf16), jnp.tile(b3, 4).reshape(1, 4 * H),
      _bd4(w4).astype(bf16), jnp.tile(b4, 4).reshape(1, 4 * H))
    return out


def _tc_node(a0, a1, x0, c_st, h_st, tp, post_ws, lstm_w, lstm_b, w1s, w1d, wo, bo):
    (q0, qb0), (q1, qb1), (q2, qb2), (q3, qb3) = post_ws
    nodeblk = pl.BlockSpec((NB, H), lambda i: (i, 0))
    specs = [nodeblk] * 5 + [pl.BlockSpec((1, 1, NB), lambda i: (i, 0, 0))]
    specs += [_full((H, H)), _full((H, H)), _full((1, H)),
              _full((H, H)), _full((1, H)), _full((H, H)), _full((1, H)),
              _full((H, H)), _full((1, H)),
              _full((H, 4 * H)), _full((H, 4 * H)), _full((1, 4 * H)),
              _full((H, H)), _full((H, H)),
              _full((H, NCLS)), _full((1, NCLS))]
    out = pl.pallas_call(
        _node_body,
        grid=(GN,),
        in_specs=specs,
        out_specs=[nodeblk] * 4 + [pl.BlockSpec((1, 1), lambda i: (0, 0))],
        out_shape=[jax.ShapeDtypeStruct((NP, H), f32)] * 4
        + [jax.ShapeDtypeStruct((1, 1), f32)],
        compiler_params=pltpu.CompilerParams(
            dimension_semantics=("arbitrary",)),
    )(a0, a1, x0, c_st, h_st, tp,
      q0[:H], q0[H:], qb0.reshape(1, H),
      q1, qb1.reshape(1, H), q2, qb2.reshape(1, H), q3, qb3.reshape(1, H),
      lstm_w[:H], lstm_w[H:], lstm_b.reshape(1, 4 * H),
      w1s, w1d, wo, bo.reshape(1, NCLS))
    return out


def kernel(x, edge_index, edge_attr, targets, params):
    del edge_attr
    p = params
    src = edge_index[0].astype(i32)
    dst = edge_index[1].astype(i32)
    pad = jnp.full((EP - E,), N, i32)
    src_p = jnp.concatenate([src, pad])
    dst_p = jnp.concatenate([dst, pad])
    src3 = src_p.reshape(NW, NCH, CG)
    xp = jnp.pad(x, ((0, NP - N), (0, 0)))
    tp = jnp.pad(targets.astype(i32), (0, NP - N)).reshape(GN, 1, NB)
    zeros_np = jnp.zeros((NP, H), f32)

    w1 = p["msg"][0][0]
    w1s = w1[:H]
    w1d = w1[H:2 * H]

    x0, a_t, b_t = _tc_pre(xp, p["pre"], w1s, w1d)
    c_st = jnp.zeros((NP, H), f32)
    h_st = jnp.zeros((NP, H), f32)
    losses = []
    for _ in range(STEPS):
        ga, gb = _sc_gather(a_t, b_t, src_p, dst_p)
        msg_p = _tc_edge(ga.reshape(EPP, 4 * H), gb.reshape(EPP, 4 * H), p["msg"])
        agg0, agg1 = _sc_scatter(msg_p.reshape(EP, H), src3, zeros_np)
        c_st, h_st, a_t, b_t, lsum = _tc_node(
            agg0, agg1, x0, c_st, h_st, tp, p["post"],
            p["lstm_W"], p["lstm_b"], w1s, w1d, p["out_W"], p["out_b"])
        losses.append(lsum[0, 0] / (N * jnp.log(2.0)))
    return jnp.mean(jnp.stack(losses))
